# baseline (speedup 1.0000x reference)
import jax
import jax.numpy as jnp
from jax import lax
from jax.experimental import pallas as pl
from jax.experimental.pallas import tpu as pltpu
from jax.experimental.pallas import tpu_sc as plsc

_N_USERS = 4096
_N_ITEMS = 4096
_B = 64
_DEGREE = 16

_NC, _NS, _L = 2, 16, 16
_NNZ_PAD = 180224
_ROWS128 = _NNZ_PAD // 128
_RT = _ROWS128 // _NS
_SLAB = 1 << 20
_PASSES = (_N_USERS * _N_ITEMS) // (_SLAB * _NC)
_SHARE = _SLAB // _NS
_ZCHUNK = 16384
_UBLK = 512
_J = _N_USERS // _UBLK


def _densify_body(rows_hbm, cols_hbm, vals_hbm, out_hbm,
                  lin_v, cols_v, vals_v, idx_v, zbuf, slab, sem):
    c = lax.axis_index("c")
    s = lax.axis_index("s")

    pltpu.sync_copy(rows_hbm.at[pl.ds(s * _RT, _RT)], lin_v)
    pltpu.sync_copy(cols_hbm.at[pl.ds(s * _RT, _RT)], cols_v)
    pltpu.sync_copy(vals_hbm.at[pl.ds(s * _RT, _RT)], vals_v)

    def _zero_zbuf(i, carry):
        zbuf[pl.ds(i * _L, _L)] = jnp.zeros((_L,), jnp.float32)
        return carry

    lax.fori_loop(0, _ZCHUNK // _L, _zero_zbuf, 0)

    garbage = _SLAB + lax.iota(jnp.int32, _L) * _L + s

    def _linify(r, carry):
        for q in range(128 // _L):
            sl = pl.ds(q * _L, _L)
            lin_v[r, sl] = lin_v[r, sl] * _N_ITEMS + cols_v[r, sl]
        return carry

    lax.fori_loop(0, _RT, _linify, 0)

    for p in range(_PASSES):
        base = (_NC * p + c) * _SLAB

        for z in range(_SHARE // _ZCHUNK):
            pltpu.sync_copy(
                zbuf, slab.at[pl.ds(s * _SHARE + z * _ZCHUNK, _ZCHUNK)])
        plsc.subcore_barrier()

        def _fire(r, carry):
            for q in range(128 // _L):
                sl = pl.ds(q * _L, _L)
                off = lin_v[r, sl] - base
                in_range = (off >= 0) & (off < _SLAB)
                idx_v[r, sl] = jnp.where(in_range, off, garbage)
            pltpu.async_copy(vals_v.at[r], slab.at[idx_v.at[r]], sem,
                             add=True)
            return carry

        lax.fori_loop(0, _RT, _fire, 0)

        def _drain(r, carry):
            pltpu.make_async_copy(
                vals_v.at[r], slab.at[idx_v.at[r]], sem).wait()
            return carry

        lax.fori_loop(0, _RT, _drain, 0)
        plsc.subcore_barrier()

        pltpu.sync_copy(slab.at[pl.ds(s * _SHARE, _SHARE)],
                        out_hbm.at[pl.ds(base + s * _SHARE, _SHARE)])
        plsc.subcore_barrier()


def _densify(rows2d, cols2d, vals2d):
    return pl.kernel(
        _densify_body,
        out_type=jax.ShapeDtypeStruct((_N_USERS * _N_ITEMS,), jnp.float32),
        mesh=plsc.VectorSubcoreMesh(
            core_axis_name="c", subcore_axis_name="s",
            num_cores=_NC, num_subcores=_NS),
        scratch_types=[
            pltpu.VMEM((_RT, 128), jnp.int32),
            pltpu.VMEM((_RT, 128), jnp.int32),
            pltpu.VMEM((_RT, 128), jnp.float32),
            pltpu.VMEM((_RT, 128), jnp.int32),
            pltpu.VMEM((_ZCHUNK,), jnp.float32),
            pltpu.VMEM_SHARED((_SLAB + _L * _L + _NS,), jnp.float32),
            pltpu.SemaphoreType.DMA,
        ],
    )(rows2d, cols2d, vals2d)


def _split_stack(t):
    hi = t.astype(jnp.bfloat16)
    lo = (t - hi.astype(jnp.float32)).astype(jnp.bfloat16)
    return jnp.concatenate([hi, lo], axis=0)


def _merge(hl):
    return hl[:_B, :] + hl[_B:, :]


def _cheby_body(scal_ref, x_ref, v_ref, out_ref,
                x_vmem, t_prev_s, t_cur_s, t_hl_s, gv_s):
    k = pl.program_id(0)
    j = pl.program_id(1)
    jrow = pl.multiple_of(j * _UBLK, _UBLK)

    @pl.when((k == 0) & (j == 0))
    def _init():
        v = v_ref[...]
        t_prev_s[...] = v
        t_cur_s[...] = v
        t_hl_s[...] = _split_stack(v)
        out_ref[...] = scal_ref[0] * v

    @pl.when(k == 0)
    def _cache_block():
        x_vmem[pl.ds(jrow, _UBLK), :] = x_ref[...].astype(jnp.bfloat16)

    x_blk = x_vmem[pl.ds(jrow, _UBLK), :]
    xv_hl = jax.lax.dot_general(
        t_hl_s[...], x_blk, (((1,), (1,)), ((), ())),
        preferred_element_type=jnp.float32)
    xv2 = _split_stack(_merge(xv_hl))
    g_hl = jax.lax.dot_general(
        xv2, x_blk, (((1,), (0,)), ((), ())),
        preferred_element_type=jnp.float32)
    g = _merge(g_hl)

    @pl.when(j == 0)
    def _store():
        gv_s[...] = g

    @pl.when(j > 0)
    def _accum():
        gv_s[...] += g

    @pl.when(j == _J - 1)
    def _finish():
        t_mid = scal_ref[17]
        inv_half = 1.0 / scal_ref[18]
        u = (gv_s[...] - t_mid * t_cur_s[...]) * inv_half
        alpha = jnp.where(k == 0, 1.0, 2.0)
        beta = jnp.where(k == 0, 0.0, 1.0)
        t_next = alpha * u - beta * t_prev_s[...]
        out_ref[...] += scal_ref[k + 1] * t_next
        t_prev_s[...] = t_cur_s[...]
        t_cur_s[...] = t_next
        t_hl_s[...] = _split_stack(t_next)


def _cheby_call(scal, dense_f32, x):
    return pl.pallas_call(
        _cheby_body,
        grid=(_DEGREE, _J),
        out_shape=jax.ShapeDtypeStruct((_B, _N_ITEMS), jnp.float32),
        in_specs=[
            pl.BlockSpec(memory_space=pltpu.SMEM),
            pl.BlockSpec((_UBLK, _N_ITEMS),
                         lambda k, j: (jnp.where(k == 0, j, 0), 0)),
            pl.BlockSpec((_B, _N_ITEMS), lambda k, j: (0, 0)),
        ],
        out_specs=pl.BlockSpec((_B, _N_ITEMS), lambda k, j: (0, 0)),
        scratch_shapes=[
            pltpu.VMEM((_N_USERS, _N_ITEMS), jnp.bfloat16),
            pltpu.VMEM((_B, _N_ITEMS), jnp.float32),
            pltpu.VMEM((_B, _N_ITEMS), jnp.float32),
            pltpu.VMEM((2 * _B, _N_ITEMS), jnp.bfloat16),
            pltpu.VMEM((_B, _N_ITEMS), jnp.float32),
        ],
        compiler_params=pltpu.CompilerParams(
            dimension_semantics=("arbitrary", "arbitrary"),
            vmem_limit_bytes=100 * 1024 * 1024,
        ),
    )(scal, dense_f32, x)


def kernel(x, vals, cheby_coeffs, t_mid, t_half, rows, cols):
    pad = _NNZ_PAD - rows.shape[0]
    rows2d = jnp.concatenate(
        [rows, jnp.zeros((pad,), rows.dtype)]).reshape(_ROWS128, 128)
    cols2d = jnp.concatenate(
        [cols, jnp.zeros((pad,), cols.dtype)]).reshape(_ROWS128, 128)
    vals2d = jnp.concatenate(
        [vals, jnp.zeros((pad,), vals.dtype)]).reshape(_ROWS128, 128)
    dense = _densify(rows2d, cols2d, vals2d).reshape(_N_USERS, _N_ITEMS)
    scal = jnp.concatenate(
        [cheby_coeffs, jnp.stack([t_mid, t_half])]).astype(jnp.float32)
    return _cheby_call(scal, dense, x)


# speedup vs baseline: 1.0730x; 1.0730x over previous
import jax
import jax.numpy as jnp
from jax import lax
from jax.experimental import pallas as pl
from jax.experimental.pallas import tpu as pltpu
from jax.experimental.pallas import tpu_sc as plsc

_N_USERS = 4096
_N_ITEMS = 4096
_B = 64
_DEGREE = 16

_NC, _NS, _L = 2, 16, 16
_NNZ_PAD = 180224
_ROWS128 = _NNZ_PAD // 128
_RT = _ROWS128 // _NS
_SLAB = 1 << 20
_PASSES = (_N_USERS * _N_ITEMS) // (_SLAB * _NC)
_SHARE = _SLAB // _NS
_ZCHUNK = 16384
_UBLK = 2048
_J = _N_USERS // _UBLK


def _densify_body(rows_hbm, cols_hbm, vals_hbm, out_hbm,
                  lin_v, cols_v, vals_v, idx_v, zbuf, slab, sem):
    c = lax.axis_index("c")
    s = lax.axis_index("s")

    pltpu.sync_copy(rows_hbm.at[pl.ds(s * _RT, _RT)], lin_v)
    pltpu.sync_copy(cols_hbm.at[pl.ds(s * _RT, _RT)], cols_v)
    pltpu.sync_copy(vals_hbm.at[pl.ds(s * _RT, _RT)], vals_v)

    def _zero_zbuf(i, carry):
        zbuf[pl.ds(i * _L, _L)] = jnp.zeros((_L,), jnp.float32)
        return carry

    lax.fori_loop(0, _ZCHUNK // _L, _zero_zbuf, 0)

    garbage = _SLAB + lax.iota(jnp.int32, _L) * _L + s

    def _linify(r, carry):
        for q in range(128 // _L):
            sl = pl.ds(q * _L, _L)
            lin_v[r, sl] = lin_v[r, sl] * _N_ITEMS + cols_v[r, sl]
        return carry

    lax.fori_loop(0, _RT, _linify, 0)

    for p in range(_PASSES):
        base = (_NC * p + c) * _SLAB

        for z in range(_SHARE // _ZCHUNK):
            pltpu.sync_copy(
                zbuf, slab.at[pl.ds(s * _SHARE + z * _ZCHUNK, _ZCHUNK)])
        plsc.subcore_barrier()

        def _fire(r, carry):
            for q in range(128 // _L):
                sl = pl.ds(q * _L, _L)
                off = lin_v[r, sl] - base
                in_range = (off >= 0) & (off < _SLAB)
                idx_v[r, sl] = jnp.where(in_range, off, garbage)
            pltpu.async_copy(vals_v.at[r], slab.at[idx_v.at[r]], sem,
                             add=True)
            return carry

        lax.fori_loop(0, _RT, _fire, 0)

        def _drain(r, carry):
            pltpu.make_async_copy(
                vals_v.at[r], slab.at[idx_v.at[r]], sem).wait()
            return carry

        lax.fori_loop(0, _RT, _drain, 0)
        plsc.subcore_barrier()

        pltpu.sync_copy(slab.at[pl.ds(s * _SHARE, _SHARE)],
                        out_hbm.at[pl.ds(base + s * _SHARE, _SHARE)])
        plsc.subcore_barrier()


def _densify(rows2d, cols2d, vals2d):
    return pl.kernel(
        _densify_body,
        out_type=jax.ShapeDtypeStruct((_N_USERS * _N_ITEMS,), jnp.float32),
        mesh=plsc.VectorSubcoreMesh(
            core_axis_name="c", subcore_axis_name="s",
            num_cores=_NC, num_subcores=_NS),
        scratch_types=[
            pltpu.VMEM((_RT, 128), jnp.int32),
            pltpu.VMEM((_RT, 128), jnp.int32),
            pltpu.VMEM((_RT, 128), jnp.float32),
            pltpu.VMEM((_RT, 128), jnp.int32),
            pltpu.VMEM((_ZCHUNK,), jnp.float32),
            pltpu.VMEM_SHARED((_SLAB + _L * _L + _NS,), jnp.float32),
            pltpu.SemaphoreType.DMA,
        ],
    )(rows2d, cols2d, vals2d)


def _split_stack(t):
    hi = t.astype(jnp.bfloat16)
    lo = (t - hi.astype(jnp.float32)).astype(jnp.bfloat16)
    return jnp.concatenate([hi, lo], axis=0)


def _merge(hl):
    return hl[:_B, :] + hl[_B:, :]


def _cheby_body(scal_ref, x_ref, v_ref, out_ref,
                t_prev_s, t_cur_s, t_hl_s, gv_s):
    k = pl.program_id(0)
    j = pl.program_id(1)

    @pl.when((k == 0) & (j == 0))
    def _init():
        v = v_ref[...]
        t_prev_s[...] = v
        t_cur_s[...] = v
        t_hl_s[...] = _split_stack(v)
        out_ref[...] = scal_ref[0] * v

    x_blk = x_ref[...]
    xv_hl = jax.lax.dot_general(
        t_hl_s[...], x_blk, (((1,), (1,)), ((), ())),
        preferred_element_type=jnp.float32)
    xv2 = _split_stack(_merge(xv_hl))
    g_hl = jax.lax.dot_general(
        xv2, x_blk, (((1,), (0,)), ((), ())),
        preferred_element_type=jnp.float32)
    g = _merge(g_hl)

    @pl.when(j == 0)
    def _store():
        gv_s[...] = g

    @pl.when(j > 0)
    def _accum():
        gv_s[...] += g

    @pl.when(j == _J - 1)
    def _finish():
        t_mid = scal_ref[17]
        inv_half = 1.0 / scal_ref[18]
        u = (gv_s[...] - t_mid * t_cur_s[...]) * inv_half
        alpha = jnp.where(k == 0, 1.0, 2.0)
        beta = jnp.where(k == 0, 0.0, 1.0)
        t_next = alpha * u - beta * t_prev_s[...]
        out_ref[...] += scal_ref[k + 1] * t_next
        t_prev_s[...] = t_cur_s[...]
        t_cur_s[...] = t_next
        t_hl_s[...] = _split_stack(t_next)


def _cheby_call(scal, dense_bf16, x):
    return pl.pallas_call(
        _cheby_body,
        grid=(_DEGREE, _J),
        out_shape=jax.ShapeDtypeStruct((_B, _N_ITEMS), jnp.float32),
        in_specs=[
            pl.BlockSpec(memory_space=pltpu.SMEM),
            pl.BlockSpec((_UBLK, _N_ITEMS), lambda k, j: (j, 0)),
            pl.BlockSpec((_B, _N_ITEMS), lambda k, j: (0, 0)),
        ],
        out_specs=pl.BlockSpec((_B, _N_ITEMS), lambda k, j: (0, 0)),
        scratch_shapes=[
            pltpu.VMEM((_B, _N_ITEMS), jnp.float32),
            pltpu.VMEM((_B, _N_ITEMS), jnp.float32),
            pltpu.VMEM((2 * _B, _N_ITEMS), jnp.bfloat16),
            pltpu.VMEM((_B, _N_ITEMS), jnp.float32),
        ],
        compiler_params=pltpu.CompilerParams(
            dimension_semantics=("arbitrary", "arbitrary"),
            vmem_limit_bytes=100 * 1024 * 1024,
        ),
    )(scal, dense_bf16, x)


def kernel(x, vals, cheby_coeffs, t_mid, t_half, rows, cols):
    pad = _NNZ_PAD - rows.shape[0]
    rows2d = jnp.concatenate(
        [rows, jnp.zeros((pad,), rows.dtype)]).reshape(_ROWS128, 128)
    cols2d = jnp.concatenate(
        [cols, jnp.zeros((pad,), cols.dtype)]).reshape(_ROWS128, 128)
    vals2d = jnp.concatenate(
        [vals, jnp.zeros((pad,), vals.dtype)]).reshape(_ROWS128, 128)
    dense = _densify(rows2d, cols2d, vals2d).reshape(_N_USERS, _N_ITEMS)
    scal = jnp.concatenate(
        [cheby_coeffs, jnp.stack([t_mid, t_half])]).astype(jnp.float32)
    return _cheby_call(scal, dense.astype(jnp.bfloat16), x)


# speedup vs baseline: 1.0747x; 1.0016x over previous
import jax
import jax.numpy as jnp
from jax import lax
from jax.experimental import pallas as pl
from jax.experimental.pallas import tpu as pltpu
from jax.experimental.pallas import tpu_sc as plsc

_N_USERS = 4096
_N_ITEMS = 4096
_B = 64
_DEGREE = 16

_NC, _NS, _L = 2, 16, 16
_NNZ_PAD = 180224
_ROWS128 = _NNZ_PAD // 128
_RT = _ROWS128 // _NS
_SLAB = 1 << 20
_PASSES = (_N_USERS * _N_ITEMS) // (_SLAB * _NC)
_SHARE = _SLAB // _NS
_ZCHUNK = 16384
_UBLK = 2048
_J = _N_USERS // _UBLK


def _densify_body(rows_hbm, cols_hbm, vals_hbm, out_hbm,
                  lin_v, cols_v, vals_v, idx_v, zbuf, slab, sem):
    c = lax.axis_index("c")
    s = lax.axis_index("s")

    pltpu.sync_copy(rows_hbm.at[pl.ds(s * _RT, _RT)], lin_v)
    pltpu.sync_copy(cols_hbm.at[pl.ds(s * _RT, _RT)], cols_v)
    pltpu.sync_copy(vals_hbm.at[pl.ds(s * _RT, _RT)], vals_v)

    def _zero_zbuf(i, carry):
        zbuf[pl.ds(i * _L, _L)] = jnp.zeros((_L,), jnp.float32)
        return carry

    lax.fori_loop(0, _ZCHUNK // _L, _zero_zbuf, 0)

    garbage = _SLAB + lax.iota(jnp.int32, _L) * _L + s

    def _linify(r, carry):
        for q in range(128 // _L):
            sl = pl.ds(q * _L, _L)
            lin_v[r, sl] = lin_v[r, sl] * _N_ITEMS + cols_v[r, sl]
        return carry

    lax.fori_loop(0, _RT, _linify, 0)

    for p in range(_PASSES):
        base = (_NC * p + c) * _SLAB

        for z in range(_SHARE // _ZCHUNK):
            pltpu.async_copy(
                zbuf, slab.at[pl.ds(s * _SHARE + z * _ZCHUNK, _ZCHUNK)], sem)

        def _mkidx(r, carry):
            for q in range(128 // _L):
                sl = pl.ds(q * _L, _L)
                off = lin_v[r, sl] - base
                in_range = (off >= 0) & (off < _SLAB)
                idx_v[r, sl] = jnp.where(in_range, off, garbage)
            return carry

        lax.fori_loop(0, _RT, _mkidx, 0)

        for z in range(_SHARE // _ZCHUNK):
            pltpu.make_async_copy(
                zbuf, slab.at[pl.ds(s * _SHARE + z * _ZCHUNK, _ZCHUNK)],
                sem).wait()
        plsc.subcore_barrier()

        def _fire(r, carry):
            pltpu.async_copy(vals_v.at[r], slab.at[idx_v.at[r]], sem,
                             add=True)
            return carry

        lax.fori_loop(0, _RT, _fire, 0)

        def _drain(r, carry):
            pltpu.make_async_copy(
                vals_v.at[r], slab.at[idx_v.at[r]], sem).wait()
            return carry

        lax.fori_loop(0, _RT, _drain, 0)
        plsc.subcore_barrier()

        pltpu.sync_copy(slab.at[pl.ds(s * _SHARE, _SHARE)],
                        out_hbm.at[pl.ds(base + s * _SHARE, _SHARE)])
        plsc.subcore_barrier()


def _densify(rows2d, cols2d, vals2d):
    return pl.kernel(
        _densify_body,
        out_type=jax.ShapeDtypeStruct((_N_USERS * _N_ITEMS,), jnp.float32),
        mesh=plsc.VectorSubcoreMesh(
            core_axis_name="c", subcore_axis_name="s",
            num_cores=_NC, num_subcores=_NS),
        scratch_types=[
            pltpu.VMEM((_RT, 128), jnp.int32),
            pltpu.VMEM((_RT, 128), jnp.int32),
            pltpu.VMEM((_RT, 128), jnp.float32),
            pltpu.VMEM((_RT, 128), jnp.int32),
            pltpu.VMEM((_ZCHUNK,), jnp.float32),
            pltpu.VMEM_SHARED((_SLAB + _L * _L + _NS,), jnp.float32),
            pltpu.SemaphoreType.DMA,
        ],
    )(rows2d, cols2d, vals2d)


def _split_stack(t):
    hi = t.astype(jnp.bfloat16)
    lo = (t - hi.astype(jnp.float32)).astype(jnp.bfloat16)
    return jnp.concatenate([hi, lo], axis=0)


def _merge(hl):
    return hl[:_B, :] + hl[_B:, :]


def _cheby_body(scal_ref, x_ref, v_ref, out_ref,
                t_prev_s, t_cur_s, t_hl_s, gv_s):
    k = pl.program_id(0)
    j = pl.program_id(1)

    @pl.when((k == 0) & (j == 0))
    def _init():
        v = v_ref[...]
        t_prev_s[...] = v
        t_cur_s[...] = v
        t_hl_s[...] = _split_stack(v)
        out_ref[...] = scal_ref[0] * v

    x_blk = x_ref[...]
    xv_hl = jax.lax.dot_general(
        t_hl_s[...], x_blk, (((1,), (1,)), ((), ())),
        preferred_element_type=jnp.float32)
    xv2 = _split_stack(_merge(xv_hl))
    g_hl = jax.lax.dot_general(
        xv2, x_blk, (((1,), (0,)), ((), ())),
        preferred_element_type=jnp.float32)
    g = _merge(g_hl)

    @pl.when(j == 0)
    def _store():
        gv_s[...] = g

    @pl.when(j > 0)
    def _accum():
        gv_s[...] += g

    @pl.when(j == _J - 1)
    def _finish():
        t_mid = scal_ref[17]
        inv_half = 1.0 / scal_ref[18]
        u = (gv_s[...] - t_mid * t_cur_s[...]) * inv_half
        alpha = jnp.where(k == 0, 1.0, 2.0)
        beta = jnp.where(k == 0, 0.0, 1.0)
        t_next = alpha * u - beta * t_prev_s[...]
        out_ref[...] += scal_ref[k + 1] * t_next
        t_prev_s[...] = t_cur_s[...]
        t_cur_s[...] = t_next
        t_hl_s[...] = _split_stack(t_next)


def _cheby_call(scal, dense_bf16, x):
    return pl.pallas_call(
        _cheby_body,
        grid=(_DEGREE, _J),
        out_shape=jax.ShapeDtypeStruct((_B, _N_ITEMS), jnp.float32),
        in_specs=[
            pl.BlockSpec(memory_space=pltpu.SMEM),
            pl.BlockSpec((_UBLK, _N_ITEMS), lambda k, j: (j, 0)),
            pl.BlockSpec((_B, _N_ITEMS), lambda k, j: (0, 0)),
        ],
        out_specs=pl.BlockSpec((_B, _N_ITEMS), lambda k, j: (0, 0)),
        scratch_shapes=[
            pltpu.VMEM((_B, _N_ITEMS), jnp.float32),
            pltpu.VMEM((_B, _N_ITEMS), jnp.float32),
            pltpu.VMEM((2 * _B, _N_ITEMS), jnp.bfloat16),
            pltpu.VMEM((_B, _N_ITEMS), jnp.float32),
        ],
        compiler_params=pltpu.CompilerParams(
            dimension_semantics=("arbitrary", "arbitrary"),
            vmem_limit_bytes=100 * 1024 * 1024,
        ),
    )(scal, dense_bf16, x)


def kernel(x, vals, cheby_coeffs, t_mid, t_half, rows, cols):
    pad = _NNZ_PAD - rows.shape[0]
    rows2d = jnp.concatenate(
        [rows, jnp.zeros((pad,), rows.dtype)]).reshape(_ROWS128, 128)
    cols2d = jnp.concatenate(
        [cols, jnp.zeros((pad,), cols.dtype)]).reshape(_ROWS128, 128)
    vals2d = jnp.concatenate(
        [vals, jnp.zeros((pad,), vals.dtype)]).reshape(_ROWS128, 128)
    dense = _densify(rows2d, cols2d, vals2d).reshape(_N_USERS, _N_ITEMS)
    scal = jnp.concatenate(
        [cheby_coeffs, jnp.stack([t_mid, t_half])]).astype(jnp.float32)
    return _cheby_call(scal, dense.astype(jnp.bfloat16), x)
